# baseline (speedup 1.0000x reference)
import functools

import jax
import jax.numpy as jnp
from jax import lax
from jax.experimental import pallas as pl
from jax.experimental.pallas import tpu as pltpu
from jax.experimental.pallas import tpu_sc as plsc

_L = 16
_C = 16
_NW = 32


def _sc_body(x_hbm, g_hbm, w_hbm, m_hbm, y_hbm, out_hbm,
             xbuf, gbuf, wbuf, mbuf, ybuf, idxw, idxm, idxy,
             sem_in0, sem_in1, sem_out0, sem_out1):
    wid = lax.axis_index("s") * 2 + lax.axis_index("c")
    n_rows = x_hbm.shape[0]
    rows_per_w = n_rows // _NW
    s_len = n_rows // 4
    base = wid * rows_per_w
    s0 = base % s_len
    n_chunks = rows_per_w // _C
    wn = w_hbm.shape[0]
    mn = m_hbm.shape[0]
    yn = y_hbm.shape[0]
    sems_in = (sem_in0, sem_in1)
    sems_out = (sem_out0, sem_out1)

    def issue_in(g, sl):
        row0 = base + g * _C
        sb = s0 + g * _C
        sem = sems_in[sl]
        for t in range(_C // _L):
            pos = lax.iota(jnp.int32, _L) + (sb + t * _L)
            idxw[sl, pl.ds(t * _L, _L)] = pos % wn
            idxm[sl, pl.ds(t * _L, _L)] = pos % mn
            idxy[sl, pl.ds(t * _L, _L)] = pos % yn
        pltpu.async_copy(x_hbm.at[pl.ds(row0, _C)], xbuf.at[sl], sem)
        pltpu.async_copy(g_hbm.at[pl.ds(sb, _C)], gbuf.at[sl], sem)
        pltpu.async_copy(w_hbm.at[idxw.at[sl]], wbuf.at[sl], sem)
        pltpu.async_copy(m_hbm.at[idxm.at[sl]], mbuf.at[sl], sem)
        pltpu.async_copy(y_hbm.at[idxy.at[sl]], ybuf.at[sl], sem)

    def wait_in(sl):
        sem = sems_in[sl]
        pltpu.make_async_copy(x_hbm.at[pl.ds(0, _C)], xbuf.at[sl], sem).wait()
        pltpu.make_async_copy(g_hbm.at[pl.ds(0, _C)], gbuf.at[sl], sem).wait()
        pltpu.make_async_copy(w_hbm.at[idxw.at[sl]], wbuf.at[sl], sem).wait()
        pltpu.make_async_copy(m_hbm.at[idxm.at[sl]], mbuf.at[sl], sem).wait()
        pltpu.make_async_copy(y_hbm.at[idxy.at[sl]], ybuf.at[sl], sem).wait()

    def wait_out(sl):
        pltpu.make_async_copy(xbuf.at[sl], out_hbm.at[pl.ds(0, _C)],
                              sems_out[sl]).wait()

    issue_in(0, 0)

    def pair(p, _):
        for sl in (0, 1):
            g = 2 * p + sl
            nxt = g + 1
            nsl = 1 - sl

            @pl.when(jnp.logical_and(nxt < n_chunks, nxt >= 2))
            def _():
                wait_out(nsl)

            @pl.when(nxt < n_chunks)
            def _():
                issue_in(nxt, nsl)

            wait_in(sl)
            pltpu.async_copy(xbuf.at[sl], out_hbm.at[pl.ds(base + g * _C, _C)],
                             sems_out[sl])
        return 0

    lax.fori_loop(0, n_chunks // 2, pair, 0, unroll=False)
    wait_out(0)
    wait_out(1)


@jax.jit
def kernel(x, global_pe, week_pe, month_pe, year_pe):
    B, S, D = x.shape
    d_g = global_pe.shape[1]
    x2 = x.reshape(B * S, D)
    mesh = plsc.VectorSubcoreMesh(core_axis_name="c", subcore_axis_name="s")
    k = functools.partial(
        pl.kernel,
        mesh=mesh,
        out_type=jax.ShapeDtypeStruct((B * S, D), jnp.float32),
        scratch_types=[
            pltpu.VMEM((2, _C, D), jnp.float32),
            pltpu.VMEM((2, _C, d_g), jnp.float32),
            pltpu.VMEM((2, _C, d_g), jnp.float32),
            pltpu.VMEM((2, _C, d_g), jnp.float32),
            pltpu.VMEM((2, _C, d_g), jnp.float32),
            pltpu.VMEM((2, _C), jnp.int32),
            pltpu.VMEM((2, _C), jnp.int32),
            pltpu.VMEM((2, _C), jnp.int32),
            pltpu.SemaphoreType.DMA,
            pltpu.SemaphoreType.DMA,
            pltpu.SemaphoreType.DMA,
            pltpu.SemaphoreType.DMA,
        ],
    )(_sc_body)
    out = k(x2, global_pe, week_pe, month_pe, year_pe)
    return out.reshape(B, S, D)


# speedup vs baseline: 2.7180x; 2.7180x over previous
import functools

import jax
import jax.numpy as jnp
from jax import lax
from jax.experimental import pallas as pl
from jax.experimental.pallas import tpu as pltpu
from jax.experimental.pallas import tpu_sc as plsc

_L = 16
_C = 16
_NW = 32


def _sc_body(x_hbm, g_hbm, w_hbm, m_hbm, y_hbm, out_hbm,
             xbuf, gbuf, wtab, mtab, ytab,
             sem_tab, sem_in0, sem_in1, sem_out0, sem_out1):
    wid = lax.axis_index("s") * 2 + lax.axis_index("c")
    n_rows = x_hbm.shape[0]
    rows_per_w = n_rows // _NW
    s_len = n_rows // 4
    base = wid * rows_per_w
    s0 = base % s_len
    n_chunks = rows_per_w // _C

    wn = w_hbm.shape[0]
    mn = m_hbm.shape[0]
    yn = y_hbm.shape[0]
    d_g = g_hbm.shape[1]

    sems_in = (sem_in0, sem_in1)
    sems_out = (sem_out0, sem_out1)

    cw = pltpu.async_copy(w_hbm, wtab, sem_tab)
    cm = pltpu.async_copy(m_hbm, mtab, sem_tab)
    cy = pltpu.async_copy(y_hbm, ytab, sem_tab)

    def issue_in(g, sl):
        row0 = base + g * _C
        sb = s0 + g * _C
        sem = sems_in[sl]
        pltpu.async_copy(x_hbm.at[pl.ds(row0, _C)], xbuf.at[sl], sem)
        pltpu.async_copy(g_hbm.at[pl.ds(sb, _C)], gbuf.at[sl], sem)

    def wait_in(sl):
        sem = sems_in[sl]
        pltpu.make_async_copy(x_hbm.at[pl.ds(0, _C)], xbuf.at[sl], sem).wait()
        pltpu.make_async_copy(g_hbm.at[pl.ds(0, _C)], gbuf.at[sl], sem).wait()

    def wait_out(sl):
        pltpu.make_async_copy(
            xbuf.at[sl], out_hbm.at[pl.ds(0, _C)], sems_out[sl]
        ).wait()

    def compute(sl, sb):
        @plsc.parallel_loop(0, _C)
        def row(j):
            s = sb + j
            widx = s % wn
            midx = s % mn
            yidx = s % yn
            for k in range(d_g // _L):
                c = pl.ds(k * _L, _L)
                xbuf[sl, j, pl.ds(0 * d_g + k * _L, _L)] = (
                    xbuf[sl, j, pl.ds(0 * d_g + k * _L, _L)] + gbuf[sl, j, c]
                )
                xbuf[sl, j, pl.ds(1 * d_g + k * _L, _L)] = (
                    xbuf[sl, j, pl.ds(1 * d_g + k * _L, _L)] + wtab[widx, c]
                )
                xbuf[sl, j, pl.ds(2 * d_g + k * _L, _L)] = (
                    xbuf[sl, j, pl.ds(2 * d_g + k * _L, _L)] + mtab[midx, c]
                )
                xbuf[sl, j, pl.ds(3 * d_g + k * _L, _L)] = (
                    xbuf[sl, j, pl.ds(3 * d_g + k * _L, _L)] + ytab[yidx, c]
                )

    issue_in(0, 0)
    cw.wait()
    cm.wait()
    cy.wait()

    def pair(p, _):
        for sl in (0, 1):
            g = 2 * p + sl
            nxt = g + 1
            nsl = 1 - sl

            @pl.when(jnp.logical_and(nxt < n_chunks, nxt >= 2))
            def _():
                wait_out(nsl)

            @pl.when(nxt < n_chunks)
            def _():
                issue_in(nxt, nsl)

            wait_in(sl)
            compute(sl, s0 + g * _C)
            pltpu.async_copy(
                xbuf.at[sl], out_hbm.at[pl.ds(base + g * _C, _C)], sems_out[sl]
            )
        return 0

    lax.fori_loop(0, n_chunks // 2, pair, 0, unroll=False)
    wait_out(0)
    wait_out(1)


@jax.jit
def kernel(x, global_pe, week_pe, month_pe, year_pe):
    B, S, D = x.shape
    d_g = global_pe.shape[1]
    x2 = x.reshape(B * S, D)
    mesh = plsc.VectorSubcoreMesh(core_axis_name="c", subcore_axis_name="s")
    k = functools.partial(
        pl.kernel,
        mesh=mesh,
        out_type=jax.ShapeDtypeStruct((B * S, D), jnp.float32),
        scratch_types=[
            pltpu.VMEM((2, _C, D), jnp.float32),
            pltpu.VMEM((2, _C, d_g), jnp.float32),
            pltpu.VMEM(week_pe.shape, jnp.float32),
            pltpu.VMEM(month_pe.shape, jnp.float32),
            pltpu.VMEM(year_pe.shape, jnp.float32),
            pltpu.SemaphoreType.DMA,
            pltpu.SemaphoreType.DMA,
            pltpu.SemaphoreType.DMA,
            pltpu.SemaphoreType.DMA,
            pltpu.SemaphoreType.DMA,
        ],
    )(_sc_body)
    out = k(x2, global_pe, week_pe, month_pe, year_pe)
    return out.reshape(B, S, D)


# speedup vs baseline: 2.8200x; 1.0375x over previous
import functools

import jax
import jax.numpy as jnp
from jax import lax
from jax.experimental import pallas as pl
from jax.experimental.pallas import tpu as pltpu
from jax.experimental.pallas import tpu_sc as plsc

_L = 16
_C = 16
_NW = 32


def _sc_body(x_hbm, g_hbm, w_hbm, m_hbm, y_hbm, out_hbm,
             xbuf0, xbuf1, gbuf0, gbuf1, wtab, mtab, ytab,
             sem_tab, sem_in0, sem_in1, sem_out0, sem_out1):
    xbufs = (xbuf0, xbuf1)
    gbufs = (gbuf0, gbuf1)
    wid = lax.axis_index("s") * 2 + lax.axis_index("c")
    n_rows = x_hbm.shape[0]
    rows_per_w = n_rows // _NW
    s_len = n_rows // 4
    base = wid * rows_per_w
    s0 = base % s_len
    n_chunks = rows_per_w // _C

    wn = w_hbm.shape[0]
    mn = m_hbm.shape[0]
    yn = y_hbm.shape[0]
    d_g = g_hbm.shape[1]

    sems_in = (sem_in0, sem_in1)
    sems_out = (sem_out0, sem_out1)

    cw = pltpu.async_copy(w_hbm, wtab, sem_tab)
    cm = pltpu.async_copy(m_hbm, mtab, sem_tab)
    cy = pltpu.async_copy(y_hbm, ytab, sem_tab)

    def issue_in(g, sl):
        row0 = base + g * _C
        sb = s0 + g * _C
        sem = sems_in[sl]
        pltpu.async_copy(x_hbm.at[pl.ds(row0, _C)], xbufs[sl], sem)
        pltpu.async_copy(g_hbm.at[pl.ds(sb, _C)], gbufs[sl], sem)

    def wait_in(sl):
        sem = sems_in[sl]
        pltpu.make_async_copy(x_hbm.at[pl.ds(0, _C)], xbufs[sl], sem).wait()
        pltpu.make_async_copy(g_hbm.at[pl.ds(0, _C)], gbufs[sl], sem).wait()

    def wait_out(sl):
        pltpu.make_async_copy(
            xbufs[sl], out_hbm.at[pl.ds(0, _C)], sems_out[sl]
        ).wait()

    def compute(sl, sb):
        @plsc.parallel_loop(0, _C)
        def row(j):
            s = sb + j
            widx = s % wn
            midx = s % mn
            yidx = s % yn
            for k in range(d_g // _L):
                c = pl.ds(k * _L, _L)
                xbufs[sl][j, pl.ds(0 * d_g + k * _L, _L)] = (
                    xbufs[sl][j, pl.ds(0 * d_g + k * _L, _L)] + gbufs[sl][j, c]
                )
                xbufs[sl][j, pl.ds(1 * d_g + k * _L, _L)] = (
                    xbufs[sl][j, pl.ds(1 * d_g + k * _L, _L)] + wtab[widx, c]
                )
                xbufs[sl][j, pl.ds(2 * d_g + k * _L, _L)] = (
                    xbufs[sl][j, pl.ds(2 * d_g + k * _L, _L)] + mtab[midx, c]
                )
                xbufs[sl][j, pl.ds(3 * d_g + k * _L, _L)] = (
                    xbufs[sl][j, pl.ds(3 * d_g + k * _L, _L)] + ytab[yidx, c]
                )

    issue_in(0, 0)
    cw.wait()
    cm.wait()
    cy.wait()

    def pair(p, _):
        for sl in (0, 1):
            g = 2 * p + sl
            nxt = g + 1
            nsl = 1 - sl

            @pl.when(jnp.logical_and(nxt < n_chunks, nxt >= 2))
            def _():
                wait_out(nsl)

            @pl.when(nxt < n_chunks)
            def _():
                issue_in(nxt, nsl)

            wait_in(sl)
            compute(sl, s0 + g * _C)
            pltpu.async_copy(
                xbufs[sl], out_hbm.at[pl.ds(base + g * _C, _C)], sems_out[sl]
            )
        return 0

    lax.fori_loop(0, n_chunks // 2, pair, 0, unroll=False)
    wait_out(0)
    wait_out(1)


@jax.jit
def kernel(x, global_pe, week_pe, month_pe, year_pe):
    B, S, D = x.shape
    d_g = global_pe.shape[1]
    x2 = x.reshape(B * S, D)
    mesh = plsc.VectorSubcoreMesh(core_axis_name="c", subcore_axis_name="s")
    k = functools.partial(
        pl.kernel,
        mesh=mesh,
        out_type=jax.ShapeDtypeStruct((B * S, D), jnp.float32),
        scratch_types=[
            pltpu.VMEM((_C, D), jnp.float32),
            pltpu.VMEM((_C, D), jnp.float32),
            pltpu.VMEM((_C, d_g), jnp.float32),
            pltpu.VMEM((_C, d_g), jnp.float32),
            pltpu.VMEM(week_pe.shape, jnp.float32),
            pltpu.VMEM(month_pe.shape, jnp.float32),
            pltpu.VMEM(year_pe.shape, jnp.float32),
            pltpu.SemaphoreType.DMA,
            pltpu.SemaphoreType.DMA,
            pltpu.SemaphoreType.DMA,
            pltpu.SemaphoreType.DMA,
            pltpu.SemaphoreType.DMA,
        ],
    )(_sc_body)
    out = k(x2, global_pe, week_pe, month_pe, year_pe)
    return out.reshape(B, S, D)
